# baseline (speedup 1.0000x reference)
import functools

import jax
import jax.numpy as jnp
from jax.experimental import pallas as pl
from jax.experimental.pallas import tpu as pltpu

D = 768
FF = 3072
E = 8
K = 2
T = 2048

TILE_M = 128
P = T * K + E * TILE_M
NT = P // TILE_M
TF = 768
NF = FF // TF

_BT1 = 256
_BT4 = 128


def _k1_body(x_ref, wg_ref, g_ref, b_ref, h_ref, ti_ref, tw_ref):
    x = x_ref[...]
    mu = jnp.mean(x, axis=-1, keepdims=True)
    var = jnp.mean((x - mu) ** 2, axis=-1, keepdims=True)
    h = (x - mu) / jnp.sqrt(var + 1e-5) * g_ref[...] + b_ref[...]
    h_ref[...] = h
    logits = jnp.dot(h, wg_ref[...], preferred_element_type=jnp.float32)
    m = jnp.max(logits, axis=-1, keepdims=True)
    ex = jnp.exp(logits - m)
    p = ex / jnp.sum(ex, axis=-1, keepdims=True)
    ei = jax.lax.broadcasted_iota(jnp.int32, (_BT1, E), 1)
    v1 = jnp.max(p, axis=-1, keepdims=True)
    i1 = jnp.min(jnp.where(p == v1, ei, E), axis=-1, keepdims=True)
    p2 = jnp.where(ei == i1, -1.0, p)
    v2 = jnp.max(p2, axis=-1, keepdims=True)
    i2 = jnp.min(jnp.where(p2 == v2, ei, E), axis=-1, keepdims=True)
    den = v1 + v2 + 1e-9
    ti_ref[...] = jnp.concatenate([i1, i2], axis=1)
    tw_ref[...] = jnp.concatenate([v1 / den, v2 / den], axis=1)


def _route_body(ef_ref, pos_ref, meta_ref):
    e = ef_ref[...]
    n = T * K
    se = jax.lax.broadcasted_iota(jnp.int32, (E, n), 0)
    oh = (e == se).astype(jnp.int32)
    c = oh
    s = 1
    while s < n:
        z = jnp.zeros((E, s), jnp.int32)
        c = c + jnp.concatenate([z, c[:, : n - s]], axis=1)
        s *= 2
    rank = jnp.sum(oh * c, axis=0, keepdims=True) - 1
    counts = c[:, n - 1 : n]
    psz = ((counts + (TILE_M - 1)) >> 7) << 7
    q = psz
    s = 1
    while s < E:
        z = jnp.zeros((s, 1), jnp.int32)
        q = q + jnp.concatenate([z, q[: E - s, :]], axis=0)
        s *= 2
    off = q - psz
    pos_ref[...] = jnp.sum(oh * off, axis=0, keepdims=True) + rank
    p_used = q[E - 1 : E, :]
    ti = jax.lax.broadcasted_iota(jnp.int32, (1, 64), 1) * TILE_M
    texp = jnp.zeros((1, 64), jnp.int32)
    for ee in range(1, E):
        texp = texp + (ti >= off[ee : ee + 1, :]).astype(jnp.int32)
    nv = p_used >> 7
    li = jax.lax.broadcasted_iota(jnp.int32, (1, 64), 1)
    meta_ref[...] = jnp.where(li == 40, nv, texp)


def _k3_body(meta_ref, posr_ref, wr_ref, h_ref, w1_ref, b1_ref, w2_ref,
             b2_ref, ys_ref, hs_s, ws_s):
    i = pl.program_id(0)
    f = pl.program_id(1)
    nv = meta_ref[40]
    valid = i < nv

    @pl.when(jnp.logical_not(valid) & (f == 0))
    def _zero_dead():
        ys_ref[...] = jnp.zeros_like(ys_ref)

    @pl.when(valid & (f == 0))
    def _gather():
        s_r = i * TILE_M + jax.lax.broadcasted_iota(jnp.int32, (TILE_M, 1), 0)
        g0 = (posr_ref[0:1, :] == s_r).astype(jnp.float32)
        g1 = (posr_ref[1:2, :] == s_r).astype(jnp.float32)
        hs_s[...] = jnp.dot(g0 + g1, h_ref[...],
                            preferred_element_type=jnp.float32)
        ws_s[...] = jnp.sum(g0 * wr_ref[0:1, :] + g1 * wr_ref[1:2, :],
                            axis=1, keepdims=True)

    @pl.when(valid)
    def _ffn():
        a = jnp.maximum(
            jnp.dot(hs_s[...], w1_ref[0], preferred_element_type=jnp.float32)
            + b1_ref[0], 0.0)
        contrib = jnp.dot(a, w2_ref[0], preferred_element_type=jnp.float32)

        @pl.when(f == 0)
        def _init():
            ys_ref[...] = b2_ref[0] + contrib

        @pl.when(f > 0)
        def _acc():
            ys_ref[...] = ys_ref[...] + contrib

        @pl.when(f == NF - 1)
        def _scale():
            ys_ref[...] = ys_ref[...] * ws_s[...]


def _k4_body(x_ref, pk_ref, ys_ref, o_ref):
    si = jax.lax.broadcasted_iota(jnp.int32, (_BT4, P), 1)
    a = ((pk_ref[:, 0:1] == si).astype(jnp.float32)
         + (pk_ref[:, 1:2] == si).astype(jnp.float32))
    o_ref[...] = x_ref[...] + jnp.dot(a, ys_ref[...],
                                      preferred_element_type=jnp.float32)


def kernel(x, Wg, W1, b1, W2, b2, gamma, beta):
    g2 = gamma.reshape(1, D)
    bt2 = beta.reshape(1, D)

    h, tidx, tw = pl.pallas_call(
        _k1_body,
        grid=(T // _BT1,),
        in_specs=[
            pl.BlockSpec((_BT1, D), lambda i: (i, 0)),
            pl.BlockSpec((D, E), lambda i: (0, 0)),
            pl.BlockSpec((1, D), lambda i: (0, 0)),
            pl.BlockSpec((1, D), lambda i: (0, 0)),
        ],
        out_specs=[
            pl.BlockSpec((_BT1, D), lambda i: (i, 0)),
            pl.BlockSpec((_BT1, K), lambda i: (i, 0)),
            pl.BlockSpec((_BT1, K), lambda i: (i, 0)),
        ],
        out_shape=[
            jax.ShapeDtypeStruct((T, D), jnp.float32),
            jax.ShapeDtypeStruct((T, K), jnp.int32),
            jax.ShapeDtypeStruct((T, K), jnp.float32),
        ],
    )(x, Wg, g2, bt2)

    ef = tidx.reshape(1, T * K)
    pos1, meta = pl.pallas_call(
        _route_body,
        in_specs=[pl.BlockSpec((1, T * K), lambda: (0, 0))],
        out_specs=[
            pl.BlockSpec((1, T * K), lambda: (0, 0)),
            pl.BlockSpec((1, 64), lambda: (0, 0)),
        ],
        out_shape=[
            jax.ShapeDtypeStruct((1, T * K), jnp.int32),
            jax.ShapeDtypeStruct((1, 64), jnp.int32),
        ],
    )(ef)

    posk = pos1.reshape(T, K)
    posr = posk.T
    wr = tw.T
    meta1 = meta.reshape(64)

    ys = pl.pallas_call(
        _k3_body,
        grid_spec=pltpu.PrefetchScalarGridSpec(
            num_scalar_prefetch=1,
            grid=(NT, NF),
            in_specs=[
                pl.BlockSpec((K, T), lambda i, f, m: (0, 0)),
                pl.BlockSpec((K, T), lambda i, f, m: (0, 0)),
                pl.BlockSpec((T, D), lambda i, f, m: (0, 0)),
                pl.BlockSpec((1, D, TF), lambda i, f, m: (m[i], 0, f)),
                pl.BlockSpec((1, 1, TF), lambda i, f, m: (m[i], 0, f)),
                pl.BlockSpec((1, TF, D), lambda i, f, m: (m[i], f, 0)),
                pl.BlockSpec((1, 1, D), lambda i, f, m: (m[i], 0, 0)),
            ],
            out_specs=pl.BlockSpec((TILE_M, D), lambda i, f, m: (i, 0)),
            scratch_shapes=[
                pltpu.VMEM((TILE_M, D), jnp.float32),
                pltpu.VMEM((TILE_M, 1), jnp.float32),
            ],
        ),
        out_shape=jax.ShapeDtypeStruct((P, D), jnp.float32),
        compiler_params=pltpu.CompilerParams(
            dimension_semantics=("arbitrary", "arbitrary")),
    )(meta1, posr, wr, h, W1, b1.reshape(E, 1, FF), W2, b2.reshape(E, 1, D))

    out = pl.pallas_call(
        _k4_body,
        grid=(T // _BT4,),
        in_specs=[
            pl.BlockSpec((_BT4, D), lambda i: (i, 0)),
            pl.BlockSpec((_BT4, K), lambda i: (i, 0)),
            pl.BlockSpec((P, D), lambda i: (0, 0)),
        ],
        out_specs=pl.BlockSpec((_BT4, D), lambda i: (i, 0)),
        out_shape=jax.ShapeDtypeStruct((T, D), jnp.float32),
    )(x, posk, ys)

    return out
